# baseline (speedup 1.0000x reference)
import jax
import jax.numpy as jnp
from jax.experimental import pallas as pl

N_TOK = 8192
N_CODE = 8192
DIM = 32
T_BLK = 256
GRID = N_TOK // T_BLK
C_COST = 0.25


def _vq_body(x_ref, enc_ref, e_ref, qf_ref, qst_ref, counts_ref, scal_ref):
    step = pl.program_id(0)

    @pl.when(step == 0)
    def _init():
        counts_ref[...] = jnp.zeros_like(counts_ref)
        scal_ref[...] = jnp.zeros_like(scal_ref)

    x = x_ref[...]
    enc = enc_ref[...]
    e = e_ref[...]

    q = jax.lax.dot_general(
        enc, e, (((1,), (0,)), ((), ())),
        preferred_element_type=jnp.float32)
    qf_ref[...] = q
    qst_ref[...] = x + (q - x)

    counts_ref[...] += jnp.sum(enc, axis=0)[None, :]
    lane = jax.lax.broadcasted_iota(jnp.int32, (1, 2), 1)
    scal_ref[...] += jnp.where(lane == 0, jnp.sum((q - x) ** 2), 0.0)

    @pl.when(step == GRID - 1)
    def _final():
        sq = jnp.sum(jnp.where(lane == 0, scal_ref[...], 0.0))
        loss = C_COST * sq / (N_TOK * DIM)
        p = counts_ref[...] / N_TOK
        ent = jnp.sum(p * jnp.log(p + 1e-10))
        scal_ref[...] = jnp.where(lane == 0, loss, jnp.exp(-ent))


def kernel(inputs, emb_w):
    x = jnp.transpose(inputs, (0, 2, 3, 1))
    in_shape = x.shape
    flat = x.reshape(N_TOK, DIM)

    distances = (
        jnp.sum(flat ** 2, axis=1, keepdims=True)
        + jnp.sum(emb_w ** 2, axis=1)
        - 2.0 * jnp.matmul(flat, emb_w.T)
    )
    encoding_indices = jnp.argmin(distances, axis=1)
    enc = jax.nn.one_hot(encoding_indices, N_CODE, dtype=jnp.float32)

    qf, qst, _counts, scal = pl.pallas_call(
        _vq_body,
        grid=(GRID,),
        in_specs=[
            pl.BlockSpec((T_BLK, DIM), lambda i: (i, 0)),
            pl.BlockSpec((T_BLK, N_CODE), lambda i: (i, 0)),
            pl.BlockSpec((N_CODE, DIM), lambda i: (0, 0)),
        ],
        out_specs=[
            pl.BlockSpec((T_BLK, DIM), lambda i: (i, 0)),
            pl.BlockSpec((T_BLK, DIM), lambda i: (i, 0)),
            pl.BlockSpec((1, N_CODE), lambda i: (0, 0)),
            pl.BlockSpec((1, 2), lambda i: (0, 0)),
        ],
        out_shape=[
            jax.ShapeDtypeStruct((N_TOK, DIM), jnp.float32),
            jax.ShapeDtypeStruct((N_TOK, DIM), jnp.float32),
            jax.ShapeDtypeStruct((1, N_CODE), jnp.float32),
            jax.ShapeDtypeStruct((1, 2), jnp.float32),
        ],
    )(flat, enc, emb_w)

    loss = scal[0, 0]
    perplexity = scal[0, 1]
    quantized_st = jnp.transpose(qst.reshape(in_shape), (0, 3, 1, 2))
    return loss, quantized_st, perplexity, enc, qf


# speedup vs baseline: 1.1008x; 1.1008x over previous
import jax
import jax.numpy as jnp
from jax.experimental import pallas as pl

N_TOK = 8192
N_CODE = 8192
DIM = 32
T_BLK = 256
GRID = N_TOK // T_BLK
C_COST = 0.25


def _vq_body(x_ref, idx_ref, e_ref, qf_ref, qst_ref, counts_ref, scal_ref):
    step = pl.program_id(0)

    @pl.when(step == 0)
    def _init():
        counts_ref[...] = jnp.zeros_like(counts_ref)
        scal_ref[...] = jnp.zeros_like(scal_ref)

    x = x_ref[...]
    idx = idx_ref[...]
    e = e_ref[...]

    iota = jax.lax.broadcasted_iota(jnp.int32, (T_BLK, N_CODE), 1)
    enc = (iota == idx).astype(jnp.float32)

    q = jax.lax.dot_general(
        enc, e, (((1,), (0,)), ((), ())),
        preferred_element_type=jnp.float32)
    qf_ref[...] = q
    qst_ref[...] = x + (q - x)

    counts_ref[...] += jnp.sum(enc, axis=0)[None, :]
    lane = jax.lax.broadcasted_iota(jnp.int32, (1, 2), 1)
    scal_ref[...] += jnp.where(lane == 0, jnp.sum((q - x) ** 2), 0.0)

    @pl.when(step == GRID - 1)
    def _final():
        sq = jnp.sum(jnp.where(lane == 0, scal_ref[...], 0.0))
        loss = C_COST * sq / (N_TOK * DIM)
        p = counts_ref[...] / N_TOK
        ent = jnp.sum(p * jnp.log(p + 1e-10))
        scal_ref[...] = jnp.where(lane == 0, loss, jnp.exp(-ent))


def kernel(inputs, emb_w):
    x = jnp.transpose(inputs, (0, 2, 3, 1))
    in_shape = x.shape
    flat = x.reshape(N_TOK, DIM)

    distances = (
        jnp.sum(flat ** 2, axis=1, keepdims=True)
        + jnp.sum(emb_w ** 2, axis=1)
        - 2.0 * jnp.matmul(flat, emb_w.T)
    )
    encoding_indices = jnp.argmin(distances, axis=1)
    enc = jax.nn.one_hot(encoding_indices, N_CODE, dtype=jnp.float32)

    qf, qst, _counts, scal = pl.pallas_call(
        _vq_body,
        grid=(GRID,),
        in_specs=[
            pl.BlockSpec((T_BLK, DIM), lambda i: (i, 0)),
            pl.BlockSpec((T_BLK, 1), lambda i: (i, 0)),
            pl.BlockSpec((N_CODE, DIM), lambda i: (0, 0)),
        ],
        out_specs=[
            pl.BlockSpec((T_BLK, DIM), lambda i: (i, 0)),
            pl.BlockSpec((T_BLK, DIM), lambda i: (i, 0)),
            pl.BlockSpec((1, N_CODE), lambda i: (0, 0)),
            pl.BlockSpec((1, 2), lambda i: (0, 0)),
        ],
        out_shape=[
            jax.ShapeDtypeStruct((N_TOK, DIM), jnp.float32),
            jax.ShapeDtypeStruct((N_TOK, DIM), jnp.float32),
            jax.ShapeDtypeStruct((1, N_CODE), jnp.float32),
            jax.ShapeDtypeStruct((1, 2), jnp.float32),
        ],
    )(flat, encoding_indices.astype(jnp.int32)[:, None], emb_w)

    loss = scal[0, 0]
    perplexity = scal[0, 1]
    quantized_st = jnp.transpose(qst.reshape(in_shape), (0, 3, 1, 2))
    return loss, quantized_st, perplexity, enc, qf
